# baseline (speedup 1.0000x reference)
import jax
import jax.numpy as jnp
from jax.experimental import pallas as pl
from jax.experimental.pallas import tpu as pltpu

_ALPHA = 16.0
_E = 8
_ER = 32
_T1 = 2048
_T2 = 1024
_N1 = 2
_N2 = 4


def _phased(x_ref, w_ref, b_ref, exp_ref, bcat_ref, out_ref, scaled_ref):
    j = pl.program_id(1)

    @pl.when(j < _N1)
    def _read_phase():
        y = jnp.dot(x_ref[...], w_ref[...], preferred_element_type=jnp.float32)
        ax = y[:, :_ER]
        logits = y[:, _ER:_ER + _E] + b_ref[...]
        m = jnp.max(logits, axis=-1, keepdims=True)
        ex = jnp.exp(logits - m)
        probs = ex / jnp.sum(ex, axis=-1, keepdims=True)
        probs_er = jnp.dot(probs, exp_ref[...],
                           preferred_element_type=jnp.float32)
        scaled_ref[pl.ds(j * _T1, _T1), :] = ax * probs_er

    @pl.when(j >= _N1)
    def _write_phase():
        k = j - _N1
        sc = scaled_ref[pl.ds(k * _T2, _T2), :]
        out_ref[...] = jnp.dot(sc, bcat_ref[...],
                               preferred_element_type=jnp.float32)


def kernel(x, lora_A, lora_B, router_w, router_b):
    b, s, d = x.shape
    e, _, r = lora_A.shape
    tokens = b * s
    per_core = tokens // 2

    x2 = x.reshape(tokens, d)
    a_cat = lora_A.transpose(1, 0, 2).reshape(d, e * r)
    w_fused = jnp.concatenate([a_cat, router_w], axis=1)
    b_cat = lora_B.reshape(e * r, d) * (_ALPHA / r)
    expand = jnp.repeat(jnp.eye(e, dtype=jnp.float32), r, axis=1)
    bias = router_b.reshape(1, e)

    def x_idx(c, j):
        return (c * _N1 + jnp.minimum(j, _N1 - 1), 0)

    def out_idx(c, j):
        return (c * _N2 + jnp.clip(j - _N1, 0, _N2 - 1), 0)

    out = pl.pallas_call(
        _phased,
        grid=(2, _N1 + _N2),
        in_specs=[
            pl.BlockSpec((_T1, d), x_idx),
            pl.BlockSpec((d, e * r + e), lambda c, j: (0, 0)),
            pl.BlockSpec((1, e), lambda c, j: (0, 0)),
            pl.BlockSpec((e, e * r), lambda c, j: (0, 0)),
            pl.BlockSpec((e * r, d), lambda c, j: (0, 0)),
        ],
        out_specs=pl.BlockSpec((_T2, d), out_idx),
        out_shape=jax.ShapeDtypeStruct((tokens, d), jnp.float32),
        scratch_shapes=[
            pltpu.VMEM((per_core, e * r), jnp.float32),
        ],
        compiler_params=pltpu.CompilerParams(
            dimension_semantics=("parallel", "arbitrary"),
            vmem_limit_bytes=60 * 1024 * 1024,
        ),
    )(x2, w_fused, bias, expand, b_cat)
    return out.reshape(b, s, d)


# speedup vs baseline: 1.0874x; 1.0874x over previous
import jax
import jax.numpy as jnp
from jax.experimental import pallas as pl
from jax.experimental.pallas import tpu as pltpu

_ALPHA = 16.0
_E = 8
_ER = 32
_T1 = 2048
_T2 = 1024


def _pass1(x_ref, w_ref, b_ref, exp_ref, scaled_ref):
    y = jnp.dot(x_ref[...], w_ref[...], preferred_element_type=jnp.float32)
    ax = y[:, :_ER]
    logits = y[:, _ER:_ER + _E] + b_ref[...]
    m = jnp.max(logits, axis=-1, keepdims=True)
    ex = jnp.exp(logits - m)
    probs = ex / jnp.sum(ex, axis=-1, keepdims=True)
    probs_er = jnp.dot(probs, exp_ref[...], preferred_element_type=jnp.float32)
    scaled_ref[...] = ax * probs_er


def _pass2(scaled_ref, bcat_ref, out_ref):
    out_ref[...] = jnp.dot(scaled_ref[...], bcat_ref[...],
                           preferred_element_type=jnp.float32)


def kernel(x, lora_A, lora_B, router_w, router_b):
    b, s, d = x.shape
    e, _, r = lora_A.shape
    tokens = b * s

    x2 = x.reshape(tokens, d)
    a_cat = lora_A.transpose(1, 0, 2).reshape(d, e * r)
    w_fused = jnp.concatenate([a_cat, router_w], axis=1)
    b_cat = lora_B.reshape(e * r, d) * (_ALPHA / r)
    expand = jnp.repeat(jnp.eye(e, dtype=jnp.float32), r, axis=1)
    bias = router_b.reshape(1, e)

    scaled = pl.pallas_call(
        _pass1,
        grid=(tokens // _T1,),
        in_specs=[
            pl.BlockSpec((_T1, d), lambda i: (i, 0)),
            pl.BlockSpec((d, e * r + e), lambda i: (0, 0)),
            pl.BlockSpec((1, e), lambda i: (0, 0)),
            pl.BlockSpec((e, e * r), lambda i: (0, 0)),
        ],
        out_specs=pl.BlockSpec((_T1, e * r), lambda i: (i, 0)),
        out_shape=jax.ShapeDtypeStruct((tokens, e * r), jnp.float32),
        compiler_params=pltpu.CompilerParams(
            dimension_semantics=("parallel",),
            vmem_limit_bytes=60 * 1024 * 1024,
        ),
    )(x2, w_fused, bias, expand)

    out = pl.pallas_call(
        _pass2,
        grid=(tokens // _T2,),
        in_specs=[
            pl.BlockSpec((_T2, e * r), lambda i: (i, 0)),
            pl.BlockSpec((e * r, d), lambda i: (0, 0)),
        ],
        out_specs=pl.BlockSpec((_T2, d), lambda i: (i, 0)),
        out_shape=jax.ShapeDtypeStruct((tokens, d), jnp.float32),
        compiler_params=pltpu.CompilerParams(
            dimension_semantics=("parallel",),
            vmem_limit_bytes=60 * 1024 * 1024,
        ),
    )(scaled, b_cat)
    return out.reshape(b, s, d)
